# baseline (device time: 76466 ns/iter reference)
import jax
import jax.numpy as jnp
from jax import lax
from jax.experimental import pallas as pl
from jax.experimental.pallas import tpu as pltpu

N_DEV = 8
N_GRP = 6
MASKS = ((4, 3, 1), (3, 1, 4), (1, 4, 3),
         (4, 3, 1), (3, 1, 4), (1, 4, 3))
ROFF = (0, 96, 192, 272, 352, 432)
RLEN = (96, 96, 80, 80, 80, 80)


def kernel(x, w_mat, scale_x, scale_w):
    m, k_loc = x.shape
    k_loc2, n = w_mat.shape
    assert k_loc == k_loc2
    m_per = m // N_DEV

    def body(x_ref, w_ref, sx_ref, sw_ref, out_ref,
             acc, w_bf, recv0, recv1, recv2, recv3, recv4, recv5,
             send_sems, recv_sems, ack_sem):
        recv = (recv0, recv1, recv2, recv3, recv4, recv5)
        my = lax.axis_index("i")

        w_bf[...] = w_ref[...].astype(jnp.bfloat16)

        def gemm(j):
            xs = x_ref[pl.ds(j * m_per, m_per), :].astype(jnp.bfloat16)
            return lax.dot_general(
                xs, w_bf[...], (((1,), (0,)), ((), ())),
                preferred_element_type=jnp.float32,
            )

        def send(g, slot, chunk_off, partner_mask):
            j = lax.bitwise_xor(my, chunk_off)
            partner = lax.bitwise_xor(my, partner_mask)
            rdma = pltpu.make_async_remote_copy(
                src_ref=acc.at[j, pl.ds(ROFF[g], RLEN[g]), :],
                dst_ref=recv[g].at[slot],
                send_sem=send_sems.at[g, slot],
                recv_sem=recv_sems.at[g, slot],
                device_id=(partner,),
                device_id_type=pl.DeviceIdType.MESH,
            )
            rdma.start()
            return rdma

        def wait_acc(g, slot, chunk_off):
            rdma = pltpu.make_async_remote_copy(
                src_ref=recv[g].at[slot],
                dst_ref=recv[g].at[slot],
                send_sem=send_sems.at[g, slot],
                recv_sem=recv_sems.at[g, slot],
                device_id=(my,),
                device_id_type=pl.DeviceIdType.MESH,
            )
            rdma.wait_recv()
            j = lax.bitwise_xor(my, chunk_off)
            rows = acc.at[j, pl.ds(ROFF[g], RLEN[g]), :]
            rows[...] = (
                rows[...].astype(jnp.float32)
                + recv[g][slot].astype(jnp.float32)
            ).astype(jnp.bfloat16)

        rdmas = []

        r0_deltas = [(v, v ^ w, 0, w) for (u, v, w) in MASKS]
        sends_of = {}
        for g, (u, v, w) in enumerate(MASKS):
            for slot, d in enumerate(r0_deltas[g]):
                sends_of.setdefault(u ^ d, []).append((g, slot))
        for e in (7, 2, 5, 6, 4, 3, 1):
            j = lax.bitwise_xor(my, e)
            acc[pl.ds(j, 1)] = gemm(j).astype(jnp.bfloat16)[None]
            for g, slot in sends_of[e]:
                u = MASKS[g][0]
                rdmas.append(send(g, slot, e, u))
        acc[pl.ds(my, 1)] = gemm(my).astype(jnp.bfloat16)[None]

        for g, (u, v, w) in enumerate(MASKS):
            wait_acc(g, 0, v)
        for g, (u, v, w) in enumerate(MASKS):
            wait_acc(g, 1, v ^ w)
        for g, (u, v, w) in enumerate(MASKS):
            rdmas.append(send(g, 4, v ^ w, v))
            rdmas.append(send(g, 5, v, v))
        for g, (u, v, w) in enumerate(MASKS):
            wait_acc(g, 2, 0)
        for g, (u, v, w) in enumerate(MASKS):
            wait_acc(g, 3, w)

        for g, (u, v, w) in enumerate(MASKS):
            wait_acc(g, 4, w)
            rdmas.append(send(g, 6, w, w))
        for g, (u, v, w) in enumerate(MASKS):
            wait_acc(g, 5, 0)

        scale = sx_ref[0] * sw_ref[0]
        for g, (u, v, w) in enumerate(MASKS):
            rdma = pltpu.make_async_remote_copy(
                src_ref=recv[g].at[6],
                dst_ref=recv[g].at[6],
                send_sem=send_sems.at[g, 6],
                recv_sem=recv_sems.at[g, 6],
                device_id=(my,),
                device_id_type=pl.DeviceIdType.MESH,
            )
            rdma.wait_recv()
            tot = (
                acc[pl.ds(my, 1), pl.ds(ROFF[g], RLEN[g]), :][0]
                .astype(jnp.float32)
                + recv[g][6].astype(jnp.float32)
            )
            out_ref[pl.ds(ROFF[g], RLEN[g]), :] = jnp.maximum(
                tot * scale, 0.0)

        for rdma in rdmas:
            rdma.wait_send()

        for mask in (1, 3, 4):
            pl.semaphore_signal(
                ack_sem, inc=1,
                device_id=(lax.bitwise_xor(my, mask),),
                device_id_type=pl.DeviceIdType.MESH,
            )
        pl.semaphore_wait(ack_sem, 3)

    return pl.pallas_call(
        body,
        out_shape=jax.ShapeDtypeStruct((m_per, n), jnp.float32),
        in_specs=[
            pl.BlockSpec(memory_space=pltpu.VMEM),
            pl.BlockSpec(memory_space=pltpu.VMEM),
            pl.BlockSpec(memory_space=pltpu.SMEM),
            pl.BlockSpec(memory_space=pltpu.SMEM),
        ],
        out_specs=pl.BlockSpec(memory_space=pltpu.VMEM),
        scratch_shapes=[
            pltpu.VMEM((N_DEV, m_per, n), jnp.bfloat16),
            pltpu.VMEM((k_loc, n), jnp.bfloat16),
        ] + [
            pltpu.VMEM((7, RLEN[g], n), jnp.bfloat16)
            for g in range(N_GRP)
        ] + [
            pltpu.SemaphoreType.DMA((N_GRP, 7)),
            pltpu.SemaphoreType.DMA((N_GRP, 7)),
            pltpu.SemaphoreType.REGULAR,
        ],
        compiler_params=pltpu.CompilerParams(
            vmem_limit_bytes=100 * 1024 * 1024,
        ),
    )(x, w_mat, scale_x, scale_w)


# device time: 76167 ns/iter; 1.0039x vs baseline; 1.0039x over previous
import jax
import jax.numpy as jnp
from jax import lax
from jax.experimental import pallas as pl
from jax.experimental.pallas import tpu as pltpu

N_DEV = 8
N_GRP = 6
MASKS = ((4, 3, 1), (3, 1, 4), (1, 4, 3),
         (4, 3, 1), (3, 1, 4), (1, 4, 3))
ROFF = (0, 96, 192, 272, 352, 432)
RLEN = (96, 96, 80, 80, 80, 80)
_COMM = True


def kernel(x, w_mat, scale_x, scale_w):
    m, k_loc = x.shape
    k_loc2, n = w_mat.shape
    assert k_loc == k_loc2
    m_per = m // N_DEV

    def body(x_ref, w_ref, sx_ref, sw_ref, out_ref,
             acc, w_bf, recv0, recv1, recv2, recv3, recv4, recv5,
             send_sems, recv_sems, ack_sem):
        recv = (recv0, recv1, recv2, recv3, recv4, recv5)
        my = lax.axis_index("i")

        w_bf[...] = w_ref[...].astype(jnp.float8_e4m3fn)

        def gemm(j):
            xs = x_ref[pl.ds(j * m_per, m_per), :].astype(jnp.float8_e4m3fn)
            return lax.dot_general(
                xs, w_bf[...], (((1,), (0,)), ((), ())),
                preferred_element_type=jnp.float32,
            )

        def send(g, slot, chunk_off, partner_mask):
            j = lax.bitwise_xor(my, chunk_off)
            partner = lax.bitwise_xor(my, partner_mask)
            rdma = pltpu.make_async_remote_copy(
                src_ref=acc.at[j, pl.ds(ROFF[g], RLEN[g]), :],
                dst_ref=recv[g].at[slot],
                send_sem=send_sems.at[g, slot],
                recv_sem=recv_sems.at[g, slot],
                device_id=(partner,),
                device_id_type=pl.DeviceIdType.MESH,
            )
            rdma.start()
            return rdma

        def wait_acc(g, slot, chunk_off):
            rdma = pltpu.make_async_remote_copy(
                src_ref=recv[g].at[slot],
                dst_ref=recv[g].at[slot],
                send_sem=send_sems.at[g, slot],
                recv_sem=recv_sems.at[g, slot],
                device_id=(my,),
                device_id_type=pl.DeviceIdType.MESH,
            )
            rdma.wait_recv()
            j = lax.bitwise_xor(my, chunk_off)
            rows = acc.at[j, pl.ds(ROFF[g], RLEN[g]), :]
            rows[...] = (
                rows[...].astype(jnp.float32)
                + recv[g][slot].astype(jnp.float32)
            ).astype(jnp.bfloat16)

        rdmas = []

        r0_deltas = [(v, v ^ w, 0, w) for (u, v, w) in MASKS]
        sends_of = {}
        for g, (u, v, w) in enumerate(MASKS):
            for slot, d in enumerate(r0_deltas[g]):
                sends_of.setdefault(u ^ d, []).append((g, slot))
        for e in (7, 2, 5, 6, 4, 3, 1):
            j = lax.bitwise_xor(my, e)
            acc[pl.ds(j, 1)] = gemm(j).astype(jnp.bfloat16)[None]
            for g, slot in sends_of[e]:
                u = MASKS[g][0]
                if _COMM:
                    rdmas.append(send(g, slot, e, u))
        acc[pl.ds(my, 1)] = gemm(my).astype(jnp.bfloat16)[None]

        for g, (u, v, w) in enumerate(MASKS if _COMM else ()):
            wait_acc(g, 0, v)
        for g, (u, v, w) in enumerate(MASKS if _COMM else ()):
            wait_acc(g, 1, v ^ w)
        for g, (u, v, w) in enumerate(MASKS if _COMM else ()):
            rdmas.append(send(g, 4, v ^ w, v))
            rdmas.append(send(g, 5, v, v))
        for g, (u, v, w) in enumerate(MASKS if _COMM else ()):
            wait_acc(g, 2, 0)
        for g, (u, v, w) in enumerate(MASKS if _COMM else ()):
            wait_acc(g, 3, w)

        for g, (u, v, w) in enumerate(MASKS if _COMM else ()):
            wait_acc(g, 4, w)
            rdmas.append(send(g, 6, w, w))
        for g, (u, v, w) in enumerate(MASKS if _COMM else ()):
            wait_acc(g, 5, 0)

        scale = sx_ref[0] * sw_ref[0]
        for g, (u, v, w) in enumerate(MASKS):
            if _COMM:
                rdma = pltpu.make_async_remote_copy(
                    src_ref=recv[g].at[6],
                    dst_ref=recv[g].at[6],
                    send_sem=send_sems.at[g, 6],
                    recv_sem=recv_sems.at[g, 6],
                    device_id=(my,),
                    device_id_type=pl.DeviceIdType.MESH,
                )
                rdma.wait_recv()
            tot = (
                acc[pl.ds(my, 1), pl.ds(ROFF[g], RLEN[g]), :][0]
                .astype(jnp.float32)
                + (recv[g][6].astype(jnp.float32) if _COMM else 0.0)
            )
            out_ref[pl.ds(ROFF[g], RLEN[g]), :] = jnp.maximum(
                tot * scale, 0.0)

        for rdma in rdmas:
            rdma.wait_send()

        for mask in ((1, 3, 4) if _COMM else ()):
            pl.semaphore_signal(
                ack_sem, inc=1,
                device_id=(lax.bitwise_xor(my, mask),),
                device_id_type=pl.DeviceIdType.MESH,
            )
        if _COMM:
            pl.semaphore_wait(ack_sem, 3)

    return pl.pallas_call(
        body,
        out_shape=jax.ShapeDtypeStruct((m_per, n), jnp.float32),
        in_specs=[
            pl.BlockSpec(memory_space=pltpu.VMEM),
            pl.BlockSpec(memory_space=pltpu.VMEM),
            pl.BlockSpec(memory_space=pltpu.SMEM),
            pl.BlockSpec(memory_space=pltpu.SMEM),
        ],
        out_specs=pl.BlockSpec(memory_space=pltpu.VMEM),
        scratch_shapes=[
            pltpu.VMEM((N_DEV, m_per, n), jnp.bfloat16),
            pltpu.VMEM((k_loc, n), jnp.float8_e4m3fn),
        ] + [
            pltpu.VMEM((7, RLEN[g], n), jnp.bfloat16)
            for g in range(N_GRP)
        ] + [
            pltpu.SemaphoreType.DMA((N_GRP, 7)),
            pltpu.SemaphoreType.DMA((N_GRP, 7)),
            pltpu.SemaphoreType.REGULAR,
        ],
        compiler_params=pltpu.CompilerParams(
            vmem_limit_bytes=100 * 1024 * 1024,
        ),
    )(x, w_mat, scale_x, scale_w)
